# baseline (speedup 1.0000x reference)
import functools

import jax
import jax.numpy as jnp
from jax import lax
from jax.experimental import pallas as pl
from jax.experimental.pallas import tpu as pltpu
from jax.experimental.pallas import tpu_sc as plsc

_N_EMB = 1024
_DIM = 64
_ROWS = 32 * 576
_BLK = 1152
_GRID = _ROWS // _BLK
_COMMIT = 0.25

_NC = 2
_NS = 16
_NW = _NC * _NS
_B_PER_W = _ROWS // _NW
_CH = 6
_CW = _B_PER_W // _CH


def _tc_body(x_ref, emb_ref, xsq_ref, esq_ref, idx_ref, loss_ref, perp_ref,
             counts_ref, acc_ref):
    i = pl.program_id(0)

    @pl.when(i == 0)
    def _init():
        acc_ref[0] = 0.0
        counts_ref[...] = jnp.zeros_like(counts_ref)

    x = x_ref[...].reshape(_BLK, _DIM)
    emb = emb_ref[...]
    scores = lax.dot_general(
        x.astype(jnp.bfloat16), emb.astype(jnp.bfloat16),
        (((1,), (1,)), ((), ())),
        preferred_element_type=jnp.float32)
    xsq = lax.transpose(xsq_ref[0], (1, 0))
    d2 = xsq + esq_ref[...] - 2.0 * scores
    dist = jnp.maximum(d2, 0.0)

    m = jnp.min(dist, axis=1, keepdims=True)
    ids = lax.broadcasted_iota(jnp.int32, dist.shape, 1)
    idx = jnp.min(jnp.where(dist == m, ids, _N_EMB), axis=1, keepdims=True)
    idx_ref[0] = lax.transpose(idx, (1, 0))

    onehot = (ids == idx).astype(jnp.float32)
    acc_ref[0] += jnp.sum(m)
    counts_ref[...] += jnp.sum(onehot, axis=0, keepdims=True)

    @pl.when(i == _GRID - 1)
    def _fini():
        n = jnp.float32(_ROWS)
        loss = (1.0 + _COMMIT) * acc_ref[0] / (n * _DIM)
        loss_ref[...] = loss * jnp.ones((1, 1), jnp.float32)
        p = counts_ref[...] / n
        ent = jnp.sum(p * jnp.log(p + 1e-10), axis=1, keepdims=True)
        perp_ref[...] = jnp.exp(-ent)


def _tc_call(x3d, embedding, xsq, esq):
    return pl.pallas_call(
        _tc_body,
        grid=(_GRID,),
        in_specs=[
            pl.BlockSpec((_BLK // 576, 576, _DIM), lambda i: (i, 0, 0)),
            pl.BlockSpec((_N_EMB, _DIM), lambda i: (0, 0)),
            pl.BlockSpec((1, 1, _BLK), lambda i: (i, 0, 0)),
            pl.BlockSpec((1, _N_EMB), lambda i: (0, 0)),
        ],
        out_specs=[
            pl.BlockSpec((1, 1, _BLK), lambda i: (i, 0, 0)),
            pl.BlockSpec((1, 1), lambda i: (0, 0)),
            pl.BlockSpec((1, 1), lambda i: (0, 0)),
        ],
        out_shape=[
            jax.ShapeDtypeStruct((_GRID, 1, _BLK), jnp.int32),
            jax.ShapeDtypeStruct((1, 1), jnp.float32),
            jax.ShapeDtypeStruct((1, 1), jnp.float32),
        ],
        scratch_shapes=[
            pltpu.VMEM((1, _N_EMB), jnp.float32),
            pltpu.SMEM((1,), jnp.float32),
        ],
    )(x3d, embedding, xsq, esq)


def _sc_gather(embedding, idx, out_shape):
    idx2 = idx.reshape(_NW, _CH, _CW)
    mesh = plsc.VectorSubcoreMesh(core_axis_name="c", subcore_axis_name="s")

    @functools.partial(
        pl.kernel,
        mesh=mesh,
        compiler_params=pltpu.CompilerParams(use_tc_tiling_on_sc=False),
        out_type=jax.ShapeDtypeStruct(out_shape, jnp.float32),
        scratch_types=[
            pltpu.VMEM((_CH, _CW), jnp.int32),
            pltpu.VMEM((_B_PER_W, _DIM), jnp.float32),
            pltpu.SemaphoreType.DMA,
        ],
    )
    def k(table_hbm, idx_hbm, out_hbm, idx_v, rows_v, sem):
        wid = lax.axis_index("s") * _NC + lax.axis_index("c")
        pltpu.sync_copy(idx_hbm.at[wid], idx_v)
        copies = [
            pltpu.async_copy(
                table_hbm.at[idx_v.at[j]],
                rows_v.at[pl.ds(j * _CW, _CW)],
                sem,
            )
            for j in range(_CH)
        ]
        for c in copies:
            c.wait()
        pltpu.sync_copy(rows_v, out_hbm.at[wid])

    return k(embedding, idx2)


def kernel(x, embedding):
    x_flat = x.reshape(_ROWS, _DIM)
    xsq = jnp.sum(x_flat ** 2, axis=1, keepdims=True).reshape(_GRID, 1, _BLK)
    esq = jnp.sum(embedding ** 2, axis=1)[None, :]
    idx3, loss, perp = _tc_call(x, embedding, xsq, esq)
    quantized = _sc_gather(embedding, idx3.reshape(_ROWS), x.shape)
    return quantized, loss[0, 0], perp[0, 0]


# speedup vs baseline: 1.0995x; 1.0995x over previous
import functools

import jax
import jax.numpy as jnp
from jax import lax
from jax.experimental import pallas as pl
from jax.experimental.pallas import tpu as pltpu
from jax.experimental.pallas import tpu_sc as plsc

_N_EMB = 1024
_DIM = 64
_BATCH = 32
_SEQ = 576
_ROWS = _BATCH * _SEQ
_COMMIT = 0.25

_NC = 2
_NS = 16
_NW = _NC * _NS
_B_PER_W = _ROWS // _NW
_CH = 6
_CW = _B_PER_W // _CH

_BPG = 2
_BLK = _BPG * _SEQ
_GRID = _BATCH // _BPG


def _tc_body(xt_ref, emb_ref, xsq_ref, esq_ref, idx_ref, loss_ref, perp_ref,
             counts_ref, acc_ref):
    i = pl.program_id(0)

    @pl.when(i == 0)
    def _init():
        acc_ref[0] = 0.0
        counts_ref[...] = jnp.zeros_like(counts_ref)

    x = lax.transpose(xt_ref[...], (0, 2, 1)).reshape(_BLK, _DIM)
    emb = emb_ref[...]
    scores = lax.dot_general(
        x.astype(jnp.bfloat16), emb.astype(jnp.bfloat16),
        (((1,), (1,)), ((), ())),
        preferred_element_type=jnp.float32)
    xsq = lax.transpose(xsq_ref[0], (1, 0))
    d2 = (xsq + esq_ref[...]) - 2.0 * scores
    dist = jnp.maximum(d2, 0.0)

    m = jnp.min(dist, axis=1, keepdims=True)
    ids = lax.broadcasted_iota(jnp.int32, dist.shape, 1)
    idx = jnp.min(jnp.where(dist == m, ids, _N_EMB), axis=1, keepdims=True)
    idx_ref[0] = lax.transpose(idx, (1, 0))

    onehot = (ids == idx).astype(jnp.float32)
    acc_ref[0] += jnp.sum(m)
    counts_ref[...] += jnp.sum(onehot, axis=0, keepdims=True)

    @pl.when(i == _GRID - 1)
    def _fini():
        n = jnp.float32(_ROWS)
        loss = (1.0 + _COMMIT) * acc_ref[0] / (n * _DIM)
        loss_ref[...] = loss * jnp.ones((1, 1), jnp.float32)
        p = counts_ref[...] / n
        ent = jnp.sum(p * jnp.log(p + 1e-10), axis=1, keepdims=True)
        perp_ref[...] = jnp.exp(-ent)


def _tc_call(xt, embedding, xsq, esq):
    return pl.pallas_call(
        _tc_body,
        grid=(_GRID,),
        in_specs=[
            pl.BlockSpec((_BPG, _DIM, _SEQ), lambda i: (i, 0, 0)),
            pl.BlockSpec((_N_EMB, _DIM), lambda i: (0, 0)),
            pl.BlockSpec((1, 1, _BLK), lambda i: (i, 0, 0)),
            pl.BlockSpec((1, _N_EMB), lambda i: (0, 0)),
        ],
        out_specs=[
            pl.BlockSpec((1, 1, _BLK), lambda i: (i, 0, 0)),
            pl.BlockSpec((1, 1), lambda i: (0, 0)),
            pl.BlockSpec((1, 1), lambda i: (0, 0)),
        ],
        out_shape=[
            jax.ShapeDtypeStruct((_GRID, 1, _BLK), jnp.int32),
            jax.ShapeDtypeStruct((1, 1), jnp.float32),
            jax.ShapeDtypeStruct((1, 1), jnp.float32),
        ],
        scratch_shapes=[
            pltpu.VMEM((1, _N_EMB), jnp.float32),
            pltpu.SMEM((1,), jnp.float32),
        ],
    )(xt, embedding, xsq, esq)


def _sc_gather(embedding, idx, out_shape):
    idx2 = idx.reshape(_NW, _CH, _CW)
    mesh = plsc.VectorSubcoreMesh(core_axis_name="c", subcore_axis_name="s")

    @functools.partial(
        pl.kernel,
        mesh=mesh,
        compiler_params=pltpu.CompilerParams(use_tc_tiling_on_sc=False),
        out_type=jax.ShapeDtypeStruct(out_shape, jnp.float32),
        scratch_types=[
            pltpu.VMEM((_CH, _CW), jnp.int32),
            pltpu.VMEM((_B_PER_W, _DIM), jnp.float32),
            pltpu.SemaphoreType.DMA,
        ],
    )
    def k(table_hbm, idx_hbm, out_hbm, idx_v, rows_v, sem):
        wid = lax.axis_index("s") * _NC + lax.axis_index("c")
        pltpu.sync_copy(idx_hbm.at[wid], idx_v)
        copies = [
            pltpu.async_copy(
                table_hbm.at[idx_v.at[j]],
                rows_v.at[pl.ds(j * _CW, _CW)],
                sem,
            )
            for j in range(_CH)
        ]
        for c in copies:
            c.wait()
        pltpu.sync_copy(rows_v, out_hbm.at[wid])

    return k(embedding, idx2)


def kernel(x, embedding):
    xt = jnp.transpose(x, (0, 2, 1))
    xsq = jnp.sum(x.reshape(_ROWS, _DIM) ** 2, axis=1,
                  keepdims=True).reshape(_GRID, 1, _BLK)
    esq = jnp.sum(embedding ** 2, axis=1)[None, :]
    idx3, loss, perp = _tc_call(xt, embedding, xsq, esq)
    quantized = _sc_gather(embedding, idx3.reshape(_ROWS), x.shape)
    return quantized, loss[0, 0], perp[0, 0]
